# baseline (speedup 1.0000x reference)
import jax
import jax.numpy as jnp
from jax.experimental import pallas as pl

B = 64
GS = 2048
TI = 512


def _rank_body(vt_ref, v_ref, rank_ref):
    it = pl.program_id(1)
    vi = vt_ref[0]
    vj = v_ref[0]
    gt = vj > vi
    eq = vj == vi
    i_glob = jax.lax.broadcasted_iota(jnp.int32, (TI, GS), 0) + it * TI
    j_glob = jax.lax.broadcasted_iota(jnp.int32, (TI, GS), 1)
    beats = gt | (eq & (j_glob < i_glob))
    rank_ref[0] = jnp.sum(beats.astype(jnp.int32), axis=1, keepdims=True)


def _scatter_body(rank_ref, out_ref):
    rt = pl.program_id(1)
    rank = rank_ref[0]
    r = jax.lax.broadcasted_iota(jnp.int32, (GS, TI), 1) + rt * TI
    i_iota = jax.lax.broadcasted_iota(jnp.int32, (GS, TI), 0)
    sel = jnp.where(rank == r, i_iota, 0)
    out_ref[0] = jnp.sum(sel, axis=0, keepdims=True).astype(jnp.float32)


def kernel(x):
    values = x[0, :, :, -1]
    v_i = values.reshape(B, GS, 1)
    v_j = values.reshape(B, 1, GS)

    rank = pl.pallas_call(
        _rank_body,
        grid=(B, GS // TI),
        in_specs=[
            pl.BlockSpec((1, TI, 1), lambda b, it: (b, it, 0)),
            pl.BlockSpec((1, 1, GS), lambda b, it: (b, 0, 0)),
        ],
        out_specs=pl.BlockSpec((1, TI, 1), lambda b, it: (b, it, 0)),
        out_shape=jax.ShapeDtypeStruct((B, GS, 1), jnp.int32),
    )(v_i, v_j)

    out = pl.pallas_call(
        _scatter_body,
        grid=(B, GS // TI),
        in_specs=[
            pl.BlockSpec((1, GS, 1), lambda b, rt: (b, 0, 0)),
        ],
        out_specs=pl.BlockSpec((1, 1, TI), lambda b, rt: (b, 0, rt)),
        out_shape=jax.ShapeDtypeStruct((B, 1, GS), jnp.float32),
    )(rank)
    return out.reshape(B, GS)


# speedup vs baseline: 5.0989x; 5.0989x over previous
import jax
import jax.numpy as jnp
from jax import lax
from jax.experimental import pallas as pl
from jax.experimental.pallas import tpu as pltpu
from jax.experimental.pallas import tpu_sc as plsc

B = 64
GS = 2048
L = 16
CHUNK = GS // L
NW = 32
RPW = B // NW
RADIX_BITS = 6
NBINS = 1 << RADIX_BITS
NPASS = 6
HSIZE = NBINS * L


def _desc_key(raw):
    bits = plsc.bitcast(raw, jnp.int32)
    m = jnp.where(bits >= 0, bits ^ jnp.int32(-2147483648), ~bits)
    return ~m


def _sort_body(in_hbm, out_hbm, in_v, key_a, key_b, val_a, val_b, hist):
    wid = lax.axis_index("s") * 2 + lax.axis_index("c")
    row0 = wid * RPW
    for rr in range(RPW):
        pltpu.sync_copy(in_hbm.at[row0 + rr], in_v.at[pl.ds(rr * GS, GS)])

    lane = lax.iota(jnp.int32, 16)
    base_idx = lane * CHUNK
    zeros16 = jnp.zeros((16,), jnp.int32)
    ones16 = jnp.ones((16,), jnp.int32)

    bufs = [(key_a, val_a), (key_b, val_b)]

    for p in range(NPASS):
        shift = RADIX_BITS * p
        if p == 0:
            src_key, src_val = None, None
        else:
            src_key, src_val = bufs[(p - 1) % 2]
        dst_key, dst_val = bufs[p % 2]

        def zero_body(i, c):
            for rr in range(RPW):
                hist[pl.ds(rr * HSIZE + i * L, L)] = zeros16
            return c
        lax.fori_loop(0, NBINS, zero_body, 0)

        def hist_body(k, c):
            idx = base_idx + k
            for rr in range(RPW):
                if p == 0:
                    key = _desc_key(plsc.load_gather(in_v, [idx + rr * GS]))
                else:
                    key = plsc.load_gather(src_key, [idx + rr * GS])
                d = lax.shift_right_logical(key, shift) & (NBINS - 1)
                slot = d * L + lane + rr * HSIZE
                plsc.addupdate_scatter(hist, [slot], ones16)
            return c
        lax.fori_loop(0, CHUNK, hist_body, 0)

        def scan_body(r, carry):
            new = []
            for rr in range(RPW):
                v = hist[pl.ds(rr * HSIZE + r * L, L)]
                incl = plsc.cumsum(v)
                hist[pl.ds(rr * HSIZE + r * L, L)] = (incl - v) + carry[rr]
                new.append(carry[rr] + jnp.sum(v))
            return tuple(new)
        lax.fori_loop(0, NBINS, scan_body, (jnp.int32(0),) * RPW)

        def perm_body(k, c):
            idx = base_idx + k
            for rr in range(RPW):
                if p == 0:
                    key = _desc_key(plsc.load_gather(in_v, [idx + rr * GS]))
                    val = idx.astype(jnp.float32)
                else:
                    key = plsc.load_gather(src_key, [idx + rr * GS])
                    val = plsc.load_gather(src_val, [idx + rr * GS])
                d = lax.shift_right_logical(key, shift) & (NBINS - 1)
                slot = d * L + lane + rr * HSIZE
                off = plsc.load_gather(hist, [slot])
                plsc.store_scatter(dst_key, [off + rr * GS], key)
                plsc.store_scatter(dst_val, [off + rr * GS], val)
                plsc.store_scatter(hist, [slot], off + 1)
            return c
        lax.fori_loop(0, CHUNK, perm_body, 0)

    final_val = bufs[(NPASS - 1) % 2][1]
    for rr in range(RPW):
        pltpu.sync_copy(final_val.at[pl.ds(rr * GS, GS)], out_hbm.at[row0 + rr])


def _sc_argsort(values):
    mesh = plsc.VectorSubcoreMesh(core_axis_name="c", subcore_axis_name="s")
    run = pl.kernel(
        _sort_body,
        out_type=jax.ShapeDtypeStruct((B, GS), jnp.float32),
        mesh=mesh,
        compiler_params=pltpu.CompilerParams(needs_layout_passes=False),
        scratch_types=[
            pltpu.VMEM((RPW * GS,), jnp.float32),
            pltpu.VMEM((RPW * GS,), jnp.int32),
            pltpu.VMEM((RPW * GS,), jnp.int32),
            pltpu.VMEM((RPW * GS,), jnp.float32),
            pltpu.VMEM((RPW * GS,), jnp.float32),
            pltpu.VMEM((RPW * HSIZE,), jnp.int32),
        ],
    )
    return run(values)


def kernel(x):
    values = x[0, :, :, -1]
    return _sc_argsort(values)


# speedup vs baseline: 6.7161x; 1.3172x over previous
import jax
import jax.numpy as jnp
from jax import lax
from jax.experimental import pallas as pl
from jax.experimental.pallas import tpu as pltpu
from jax.experimental.pallas import tpu_sc as plsc

B = 64
GS = 2048
L = 16
CHUNK = GS // L
NW = 32
RPW = B // NW
RADIX_BITS = 6
NBINS = 1 << RADIX_BITS
NPASS = 6
HSIZE = NBINS * L
NVREG = NBINS // L


def _desc_key(raw):
    bits = plsc.bitcast(raw, jnp.int32)
    m = jnp.where(bits >= 0, bits ^ jnp.int32(-2147483648), ~bits)
    return ~m


def _sort_body(in_hbm, out_hbm, in_v, key_a, key_b, val_a, val_b, hist, offs):
    wid = lax.axis_index("s") * 2 + lax.axis_index("c")
    row0 = wid * RPW
    for rr in range(RPW):
        pltpu.sync_copy(in_hbm.at[row0 + rr], in_v.at[pl.ds(rr * GS, GS)])

    lane = lax.iota(jnp.int32, 16)
    base_idx = lane * CHUNK
    lane_hist = lane * NBINS
    zeros16 = jnp.zeros((16,), jnp.int32)
    ones16 = jnp.ones((16,), jnp.int32)

    bufs = [(key_a, val_a), (key_b, val_b)]

    def z_body(i, c):
        hist[pl.ds(i * L, L)] = zeros16
        return c
    lax.fori_loop(0, RPW * HSIZE // L, z_body, 0, unroll=4)

    def b0_body(k, c):
        idx = base_idx + k
        for rr in range(RPW):
            key = _desc_key(plsc.load_gather(in_v, [idx + rr * GS]))
            d = key & (NBINS - 1)
            plsc.addupdate_scatter(hist, [lane_hist + d + rr * HSIZE], ones16)
        return c
    lax.fori_loop(0, CHUNK, b0_body, 0, unroll=2)

    for p in range(NPASS):
        shift = RADIX_BITS * p
        shift_next = RADIX_BITS * (p + 1)
        last = p == NPASS - 1
        if p == 0:
            src_key, src_val = None, None
        else:
            src_key, src_val = bufs[(p - 1) % 2]
        dst_key, dst_val = bufs[p % 2]

        def tot_body(l, T):
            out = []
            for rr in range(RPW):
                for j in range(NVREG):
                    h = hist[pl.ds(rr * HSIZE + l * NBINS + j * L, L)]
                    out.append(T[rr * NVREG + j] + h)
            return tuple(out)
        T = lax.fori_loop(0, L, tot_body, (zeros16,) * (RPW * NVREG),
                          unroll=2)

        R = []
        for rr in range(RPW):
            carry = jnp.int32(0)
            for j in range(NVREG):
                t = T[rr * NVREG + j]
                incl = plsc.cumsum(t)
                R.append((incl - t) + carry)
                carry = carry + jnp.sum(t)
        def run_body(l, Rc):
            out = []
            for rr in range(RPW):
                for j in range(NVREG):
                    addr = rr * HSIZE + l * NBINS + j * L
                    h = hist[pl.ds(addr, L)]
                    offs[pl.ds(addr, L)] = Rc[rr * NVREG + j]
                    hist[pl.ds(addr, L)] = zeros16
                    out.append(Rc[rr * NVREG + j] + h)
            return tuple(out)
        lax.fori_loop(0, L, run_body, tuple(R), unroll=2)

        def perm_body(k, c):
            idx = base_idx + k
            for rr in range(RPW):
                if p == 0:
                    key = _desc_key(plsc.load_gather(in_v, [idx + rr * GS]))
                    val = idx.astype(jnp.float32)
                else:
                    key = plsc.load_gather(src_key, [idx + rr * GS])
                    val = plsc.load_gather(src_val, [idx + rr * GS])
                d = lax.shift_right_logical(key, shift) & (NBINS - 1)
                slot = lane_hist + d + rr * HSIZE
                off = plsc.load_gather(offs, [slot])
                plsc.store_scatter(offs, [slot], off + 1)
                plsc.store_scatter(dst_val, [off + rr * GS], val)
                if not last:
                    plsc.store_scatter(dst_key, [off + rr * GS], key)
                    d2 = lax.shift_right_logical(key, shift_next) & (NBINS - 1)
                    slot2 = (lax.shift_right_logical(off, 7) * NBINS
                             + d2 + rr * HSIZE)
                    plsc.addupdate_scatter(hist, [slot2], ones16)
            return c
        lax.fori_loop(0, CHUNK, perm_body, 0, unroll=2)

    final_val = bufs[(NPASS - 1) % 2][1]
    for rr in range(RPW):
        pltpu.sync_copy(final_val.at[pl.ds(rr * GS, GS)], out_hbm.at[row0 + rr])


def _sc_argsort(values):
    mesh = plsc.VectorSubcoreMesh(core_axis_name="c", subcore_axis_name="s")
    run = pl.kernel(
        _sort_body,
        out_type=jax.ShapeDtypeStruct((B, GS), jnp.float32),
        mesh=mesh,
        compiler_params=pltpu.CompilerParams(needs_layout_passes=False),
        scratch_types=[
            pltpu.VMEM((RPW * GS,), jnp.float32),
            pltpu.VMEM((RPW * GS,), jnp.int32),
            pltpu.VMEM((RPW * GS,), jnp.int32),
            pltpu.VMEM((RPW * GS,), jnp.float32),
            pltpu.VMEM((RPW * GS,), jnp.float32),
            pltpu.VMEM((RPW * HSIZE,), jnp.int32),
            pltpu.VMEM((RPW * HSIZE,), jnp.int32),
        ],
    )
    return run(values)


def kernel(x):
    values = x[0, :, :, -1]
    return _sc_argsort(values)


# speedup vs baseline: 7.4597x; 1.1107x over previous
import jax
import jax.numpy as jnp
from jax import lax
from jax.experimental import pallas as pl
from jax.experimental.pallas import tpu as pltpu
from jax.experimental.pallas import tpu_sc as plsc

B = 64
GS = 2048
L = 16
CHUNK = GS // L
NW = 32
RPW = B // NW
NBINS = 64
NPASS = 6
HSIZE = NBINS * L
NVREG = NBINS // L
IDXB = 11
IMASK = (1 << IDXB) - 1
SH = (11, 17, 23, 29, 11, 17)
TRANS = 3


def _desc_key(raw):
    bits = plsc.bitcast(raw, jnp.int32)
    m = jnp.where(bits >= 0, bits ^ jnp.int32(-2147483648), ~bits)
    return ~m


def _sort_body(in_hbm, out_hbm, in_v, buf_a, buf_b, khigh, out_f, hist, offs):
    wid = lax.axis_index("s") * 2 + lax.axis_index("c")
    row0 = wid * RPW
    for rr in range(RPW):
        pltpu.sync_copy(in_hbm.at[row0 + rr], in_v.at[pl.ds(rr * GS, GS)])

    lane = lax.iota(jnp.int32, 16)
    base_idx = lane * CHUNK
    lane_hist = lane * NBINS
    zeros16 = jnp.zeros((16,), jnp.int32)
    ones16 = jnp.ones((16,), jnp.int32)

    bufs = [buf_a, buf_b]

    def z_body(i, c):
        hist[pl.ds(i * L, L)] = zeros16
        return c
    lax.fori_loop(0, RPW * HSIZE // L, z_body, 0, unroll=4)

    def b0_body(k, c):
        idx = base_idx + k
        for rr in range(RPW):
            key = _desc_key(plsc.load_gather(in_v, [idx + rr * GS]))
            d = key & (NBINS - 1)
            plsc.addupdate_scatter(hist, [lane_hist + d + rr * HSIZE], ones16)
            plsc.store_scatter(khigh, [idx + rr * GS],
                               lax.shift_right_logical(key, 21))
        return c
    lax.fori_loop(0, CHUNK, b0_body, 0, unroll=2)

    for p in range(NPASS):
        last = p == NPASS - 1
        src = bufs[(p - 1) % 2] if p > 0 else None
        dst = bufs[p % 2]

        def tot_body(l, T):
            out = []
            for rr in range(RPW):
                for j in range(NVREG):
                    h = hist[pl.ds(rr * HSIZE + l * NBINS + j * L, L)]
                    out.append(T[rr * NVREG + j] + h)
            return tuple(out)
        T = lax.fori_loop(0, L, tot_body, (zeros16,) * (RPW * NVREG),
                          unroll=2)

        R = []
        for rr in range(RPW):
            carry = jnp.int32(0)
            for j in range(NVREG):
                t = T[rr * NVREG + j]
                incl = plsc.cumsum(t)
                R.append((incl - t) + carry)
                carry = carry + jnp.sum(t)

        def run_body(l, Rc):
            out = []
            for rr in range(RPW):
                for j in range(NVREG):
                    addr = rr * HSIZE + l * NBINS + j * L
                    h = hist[pl.ds(addr, L)]
                    offs[pl.ds(addr, L)] = Rc[rr * NVREG + j]
                    hist[pl.ds(addr, L)] = zeros16
                    out.append(Rc[rr * NVREG + j] + h)
            return tuple(out)
        lax.fori_loop(0, L, run_body, tuple(R), unroll=2)

        def perm_body(k, c):
            idx = base_idx + k
            for rr in range(RPW):
                if p == 0:
                    key = _desc_key(plsc.load_gather(in_v, [idx + rr * GS]))
                    cur = lax.shift_left(key, IDXB) | idx
                else:
                    cur = plsc.load_gather(src, [idx + rr * GS])
                d = lax.shift_right_logical(cur, SH[p]) & (NBINS - 1)
                slot = lane_hist + d + rr * HSIZE
                off = plsc.load_gather(offs, [slot])
                plsc.store_scatter(offs, [slot], off + 1)
                if p == TRANS:
                    idxv = cur & IMASK
                    kh = plsc.load_gather(khigh, [idxv + rr * GS])
                    scat = lax.shift_left(kh, IDXB) | idxv
                elif last:
                    plsc.store_scatter(
                        out_f, [off + rr * GS],
                        (cur & IMASK).astype(jnp.float32))
                    continue
                else:
                    scat = cur
                plsc.store_scatter(dst, [off + rr * GS], scat)
                d2 = lax.shift_right_logical(scat, SH[p + 1]) & (NBINS - 1)
                slot2 = (lax.shift_right_logical(off, 7) * NBINS
                         + d2 + rr * HSIZE)
                plsc.addupdate_scatter(hist, [slot2], ones16)
            return c
        lax.fori_loop(0, CHUNK, perm_body, 0, unroll=2)

    for rr in range(RPW):
        pltpu.sync_copy(out_f.at[pl.ds(rr * GS, GS)], out_hbm.at[row0 + rr])


def _sc_argsort(values):
    mesh = plsc.VectorSubcoreMesh(core_axis_name="c", subcore_axis_name="s")
    run = pl.kernel(
        _sort_body,
        out_type=jax.ShapeDtypeStruct((B, GS), jnp.float32),
        mesh=mesh,
        compiler_params=pltpu.CompilerParams(needs_layout_passes=False),
        scratch_types=[
            pltpu.VMEM((RPW * GS,), jnp.float32),
            pltpu.VMEM((RPW * GS,), jnp.int32),
            pltpu.VMEM((RPW * GS,), jnp.int32),
            pltpu.VMEM((RPW * GS,), jnp.int32),
            pltpu.VMEM((RPW * GS,), jnp.float32),
            pltpu.VMEM((RPW * HSIZE,), jnp.int32),
            pltpu.VMEM((RPW * HSIZE,), jnp.int32),
        ],
    )
    return run(values)


def kernel(x):
    values = x[0, :, :, -1]
    return _sc_argsort(values)
